# initial kernel scaffold (unmeasured)
import jax
import jax.numpy as jnp
from jax import lax
from jax.experimental import pallas as pl
from jax.experimental.pallas import tpu as pltpu

N_DEV = 4
B = 2
SQ = 256
SKV = 256
HQ = 4
DH = 64
BLK = 64
D_MODEL = 512
NEG = jnp.float32(-1e9)


def kernel(x, Wq, K_ext, V_ext, Wo):
    def body(x_ref, wq_ref, k_ref, v_ref, wo_ref, out_ref,
             kvbuf, sbuf, ybuf,
             copy_sems, kv_send_sems, kv_recv_sem, y_send_sems, y_recv_sems):
        my = lax.axis_index("i")

        barrier = pltpu.get_barrier_semaphore()
        for d in range(N_DEV):
            pl.semaphore_signal(barrier, inc=1, device_id=(d,),
                                device_id_type=pl.DeviceIdType.MESH)
        pl.semaphore_wait(barrier, N_DEV)

        def kv_send(j):
            return pltpu.make_async_remote_copy(
                src_ref=sbuf.at[j - 1], dst_ref=kvbuf,
                send_sem=kv_send_sems.at[j - 1], recv_sem=kv_recv_sem,
                device_id=(j,), device_id_type=pl.DeviceIdType.MESH)

        @pl.when(my == 0)
        def _():
            slot_copies = []
            for j in range(1, N_DEV):
                for t, ref in ((0, k_ref), (1, v_ref)):
                    c = pltpu.make_async_copy(
                        ref.at[:, :, j * HQ:(j + 1) * HQ, :],
                        sbuf.at[j - 1, t],
                        copy_sems.at[2 * (j - 1) + t])
                    c.start()
                    slot_copies.append(c)
            own_copies = []
            for t, ref in ((0, k_ref), (1, v_ref)):
                c = pltpu.make_async_copy(ref.at[:, :, 0:HQ, :],
                                          kvbuf.at[t], copy_sems.at[6 + t])
                c.start()
                own_copies.append(c)
            for c in slot_copies:
                c.wait()
            for j in range(1, N_DEV):
                kv_send(j).start()
            for c in own_copies:
                c.wait()

        q = [jnp.dot(x_ref[b], wq_ref[...], preferred_element_type=jnp.float32)
             for b in range(B)]

        @pl.when(my != 0)
        def _():
            kv_send(1).wait_recv()

        rb = lax.broadcasted_iota(jnp.int32, (SQ, SKV), 0) // BLK
        cb = lax.broadcasted_iota(jnp.int32, (SQ, SKV), 1) // BLK
        mask = cb <= rb

        ys = []
        for b in range(B):
            ctxs = []
            for h in range(HQ):
                k_h = kvbuf[0, b, :, h, :]
                v_h = kvbuf[1, b, :, h, :]
                q_h = q[b][:, h * DH:(h + 1) * DH]
                s = lax.dot_general(q_h, k_h, (((1,), (1,)), ((), ())),
                                    preferred_element_type=jnp.float32)
                s = jnp.where(mask, s * jnp.float32(0.125), NEG)
                m = jnp.max(s, axis=1, keepdims=True)
                w = jnp.exp(s - m)
                p = w / jnp.sum(w, axis=1, keepdims=True)
                ctxs.append(jnp.dot(p, v_h, preferred_element_type=jnp.float32))
            ctx = jnp.concatenate(ctxs, axis=1)
            ys.append(jnp.dot(ctx, wo_ref[...],
                              preferred_element_type=jnp.float32))

        def y_rdma(src_dev, dst_dev):
            return pltpu.make_async_remote_copy(
                src_ref=ybuf.at[src_dev], dst_ref=ybuf.at[src_dev],
                send_sem=y_send_sems.at[dst_dev],
                recv_sem=y_recv_sems.at[src_dev],
                device_id=(dst_dev,), device_id_type=pl.DeviceIdType.MESH)

        for i in range(N_DEV):
            @pl.when(my == i)
            def _(i=i):
                ybuf[i, 0] = ys[0]
                ybuf[i, 1] = ys[1]
                sends = [y_rdma(i, j) for j in range(N_DEV) if j != i]
                for r in sends:
                    r.start()
                for j in range(N_DEV):
                    if j != i:
                        y_rdma(j, i).wait_recv()
                for r in sends:
                    r.wait_send()
                if i == 0:
                    for j in range(1, N_DEV):
                        kv_send(j).wait_send()

        out_ref[...] = ybuf[0] + ybuf[1] + ybuf[2] + ybuf[3]

    return pl.pallas_call(
        body,
        out_shape=jax.ShapeDtypeStruct((B, SQ, D_MODEL), jnp.float32),
        in_specs=[
            pl.BlockSpec(memory_space=pltpu.VMEM),
            pl.BlockSpec(memory_space=pltpu.VMEM),
            pl.BlockSpec(memory_space=pltpu.ANY),
            pl.BlockSpec(memory_space=pltpu.ANY),
            pl.BlockSpec(memory_space=pltpu.VMEM),
        ],
        out_specs=pl.BlockSpec(memory_space=pltpu.VMEM),
        scratch_shapes=[
            pltpu.VMEM((2, B, SKV, HQ, DH), jnp.float32),
            pltpu.VMEM((N_DEV - 1, 2, B, SKV, HQ, DH), jnp.float32),
            pltpu.VMEM((N_DEV, B, SQ, D_MODEL), jnp.float32),
            pltpu.SemaphoreType.DMA((8,)),
            pltpu.SemaphoreType.DMA((N_DEV - 1,)),
            pltpu.SemaphoreType.DMA,
            pltpu.SemaphoreType.DMA((N_DEV,)),
            pltpu.SemaphoreType.DMA((N_DEV,)),
        ],
        compiler_params=pltpu.CompilerParams(collective_id=0),
    )(x, Wq, K_ext, V_ext, Wo)


# baseline (device time: 85731 ns/iter reference)
import jax
import jax.numpy as jnp
from jax import lax
from jax.experimental import pallas as pl
from jax.experimental.pallas import tpu as pltpu

N_DEV = 4
B = 2
SQ = 256
SKV = 256
HQ = 4
DH = 64
BLK = 64
D_MODEL = 512
NEG = -1e9


def kernel(x, Wq, K_ext, V_ext, Wo):
    def body(x_ref, wq_ref, k_ref, v_ref, wo_ref, out_ref,
             kvbuf, sbuf, ybuf,
             copy_sems, kv_send_sems, kv_recv_sem, y_send_sems, y_recv_sems):
        my = lax.axis_index("i")

        barrier = pltpu.get_barrier_semaphore()
        for d in range(N_DEV):
            pl.semaphore_signal(barrier, inc=1, device_id=(d,),
                                device_id_type=pl.DeviceIdType.MESH)
        pl.semaphore_wait(barrier, N_DEV)

        def kv_send(j):
            return pltpu.make_async_remote_copy(
                src_ref=sbuf.at[j - 1], dst_ref=kvbuf,
                send_sem=kv_send_sems.at[j - 1], recv_sem=kv_recv_sem,
                device_id=(j,), device_id_type=pl.DeviceIdType.MESH)

        @pl.when(my == 0)
        def _():
            slot_copies = []
            for j in range(1, N_DEV):
                for t, ref in ((0, k_ref), (1, v_ref)):
                    c = pltpu.make_async_copy(
                        ref.at[:, :, j * HQ:(j + 1) * HQ, :],
                        sbuf.at[j - 1, t],
                        copy_sems.at[2 * (j - 1) + t])
                    c.start()
                    slot_copies.append(c)
            own_copies = []
            for t, ref in ((0, k_ref), (1, v_ref)):
                c = pltpu.make_async_copy(ref.at[:, :, 0:HQ, :],
                                          kvbuf.at[t], copy_sems.at[6 + t])
                c.start()
                own_copies.append(c)
            for c in slot_copies:
                c.wait()
            for j in range(1, N_DEV):
                kv_send(j).start()
            for c in own_copies:
                c.wait()

        q = [jnp.dot(x_ref[b], wq_ref[...], preferred_element_type=jnp.float32)
             for b in range(B)]

        @pl.when(my != 0)
        def _():
            kv_send(1).wait_recv()

        rb = lax.broadcasted_iota(jnp.int32, (SQ, SKV), 0) // BLK
        cb = lax.broadcasted_iota(jnp.int32, (SQ, SKV), 1) // BLK
        mask = cb <= rb

        ys = []
        for b in range(B):
            ctxs = []
            for h in range(HQ):
                k_h = kvbuf[0, b, :, h, :]
                v_h = kvbuf[1, b, :, h, :]
                q_h = q[b][:, h * DH:(h + 1) * DH]
                s = lax.dot_general(q_h, k_h, (((1,), (1,)), ((), ())),
                                    preferred_element_type=jnp.float32)
                s = jnp.where(mask, s * 0.125, NEG)
                m = jnp.max(s, axis=1, keepdims=True)
                w = jnp.exp(s - m)
                p = w / jnp.sum(w, axis=1, keepdims=True)
                ctxs.append(jnp.dot(p, v_h, preferred_element_type=jnp.float32))
            ctx = jnp.concatenate(ctxs, axis=1)
            ys.append(jnp.dot(ctx, wo_ref[...],
                              preferred_element_type=jnp.float32))

        def y_rdma(src_dev, dst_dev):
            return pltpu.make_async_remote_copy(
                src_ref=ybuf.at[src_dev], dst_ref=ybuf.at[src_dev],
                send_sem=y_send_sems.at[dst_dev],
                recv_sem=y_recv_sems.at[src_dev],
                device_id=(dst_dev,), device_id_type=pl.DeviceIdType.MESH)

        for i in range(N_DEV):
            @pl.when(my == i)
            def _(i=i):
                ybuf[i, 0] = ys[0]
                ybuf[i, 1] = ys[1]
                sends = [y_rdma(i, j) for j in range(N_DEV) if j != i]
                for r in sends:
                    r.start()
                for j in range(N_DEV):
                    if j != i:
                        y_rdma(j, i).wait_recv()
                for r in sends:
                    r.wait_send()
                if i == 0:
                    for j in range(1, N_DEV):
                        kv_send(j).wait_send()

        out_ref[...] = ybuf[0] + ybuf[1] + ybuf[2] + ybuf[3]

    return pl.pallas_call(
        body,
        out_shape=jax.ShapeDtypeStruct((B, SQ, D_MODEL), jnp.float32),
        in_specs=[
            pl.BlockSpec(memory_space=pltpu.VMEM),
            pl.BlockSpec(memory_space=pltpu.VMEM),
            pl.BlockSpec(memory_space=pl.ANY),
            pl.BlockSpec(memory_space=pl.ANY),
            pl.BlockSpec(memory_space=pltpu.VMEM),
        ],
        out_specs=pl.BlockSpec(memory_space=pltpu.VMEM),
        scratch_shapes=[
            pltpu.VMEM((2, B, SKV, HQ, DH), jnp.float32),
            pltpu.VMEM((N_DEV - 1, 2, B, SKV, HQ, DH), jnp.float32),
            pltpu.VMEM((N_DEV, B, SQ, D_MODEL), jnp.float32),
            pltpu.SemaphoreType.DMA((8,)),
            pltpu.SemaphoreType.DMA((N_DEV - 1,)),
            pltpu.SemaphoreType.DMA,
            pltpu.SemaphoreType.DMA((N_DEV,)),
            pltpu.SemaphoreType.DMA((N_DEV,)),
        ],
        compiler_params=pltpu.CompilerParams(collective_id=0),
    )(x, Wq, K_ext, V_ext, Wo)


# device time: 47607 ns/iter; 1.8008x vs baseline; 1.8008x over previous
import jax
import jax.numpy as jnp
from jax import lax
from jax.experimental import pallas as pl
from jax.experimental.pallas import tpu as pltpu

N_DEV = 4
B = 2
SQ = 256
SKV = 256
HQ = 4
DH = 64
BLK = 64
D_MODEL = 512
NEG = -1e9


def kernel(x, Wq, K_ext, V_ext, Wo):
    f32 = jnp.float32
    bf16 = jnp.bfloat16

    def body(x_ref, wq_ref, k_ref, v_ref, wo_ref, out_ref,
             kstage, vstage, kvbuf, sbuf, ybuf,
             copy_sems, kv_send_sems, kv_recv_sem, y_send_sems, y_recv_sems):
        my = lax.axis_index("i")

        barrier = pltpu.get_barrier_semaphore()
        for d in range(N_DEV):
            pl.semaphore_signal(barrier, inc=1, device_id=(d,),
                                device_id_type=pl.DeviceIdType.MESH)
        pl.semaphore_wait(barrier, N_DEV)

        def kv_send(j):
            return pltpu.make_async_remote_copy(
                src_ref=sbuf.at[j - 1], dst_ref=kvbuf,
                send_sem=kv_send_sems.at[j - 1], recv_sem=kv_recv_sem,
                device_id=(j,), device_id_type=pl.DeviceIdType.MESH)

        @pl.when(my == 0)
        def _():
            ck = pltpu.make_async_copy(k_ref, kstage, copy_sems.at[0])
            cv = pltpu.make_async_copy(v_ref, vstage, copy_sems.at[1])
            ck.start()
            cv.start()
            ck.wait()
            cv.wait()
            for j in range(1, N_DEV):
                sbuf[j - 1, 0] = kstage[:, :, j * HQ:(j + 1) * HQ, :].astype(bf16)
                sbuf[j - 1, 1] = vstage[:, :, j * HQ:(j + 1) * HQ, :].astype(bf16)
            for j in range(1, N_DEV):
                kv_send(j).start()
            kvbuf[0] = kstage[:, :, 0:HQ, :].astype(bf16)
            kvbuf[1] = vstage[:, :, 0:HQ, :].astype(bf16)

        wq_b = wq_ref[...].astype(bf16)
        q = [jnp.dot(x_ref[b].astype(bf16), wq_b, preferred_element_type=f32)
             for b in range(B)]

        @pl.when(my != 0)
        def _():
            kv_send(1).wait_recv()

        rb = lax.broadcasted_iota(jnp.int32, (SQ, SKV), 0) // BLK
        cb = lax.broadcasted_iota(jnp.int32, (SQ, SKV), 1) // BLK
        mask = cb <= rb
        wo_b = wo_ref[...].astype(bf16)

        def y_rdma(src_dev, dst_dev, b):
            return pltpu.make_async_remote_copy(
                src_ref=ybuf.at[src_dev, b], dst_ref=ybuf.at[src_dev, b],
                send_sem=y_send_sems.at[b, dst_dev],
                recv_sem=y_recv_sems.at[b, src_dev],
                device_id=(dst_dev,), device_id_type=pl.DeviceIdType.MESH)

        for b in range(B):
            ctxs = []
            for h in range(HQ):
                k_h = kvbuf[0, b, :, h, :]
                v_h = kvbuf[1, b, :, h, :]
                q_h = q[b][:, h * DH:(h + 1) * DH].astype(bf16)
                s = lax.dot_general(q_h, k_h, (((1,), (1,)), ((), ())),
                                    preferred_element_type=f32)
                s = jnp.where(mask, s * 0.125, NEG)
                m = jnp.max(s, axis=1, keepdims=True)
                w = jnp.exp(s - m)
                p = w / jnp.sum(w, axis=1, keepdims=True)
                ctxs.append(lax.dot_general(
                    p.astype(bf16), v_h, (((1,), (0,)), ((), ())),
                    preferred_element_type=f32))
            ctx = jnp.concatenate(ctxs, axis=1)
            y_b = jnp.dot(ctx.astype(bf16), wo_b, preferred_element_type=f32)
            for i in range(N_DEV):
                @pl.when(my == i)
                def _(i=i, y_b=y_b, b=b):
                    ybuf[i, b] = y_b.astype(bf16)
                    for j in range(N_DEV):
                        if j != i:
                            y_rdma(i, j, b).start()

        for i in range(N_DEV):
            @pl.when(my == i)
            def _(i=i):
                for b in range(B):
                    for j in range(N_DEV):
                        if j != i:
                            y_rdma(j, i, b).wait_recv()
                for b in range(B):
                    for j in range(N_DEV):
                        if j != i:
                            y_rdma(i, j, b).wait_send()
                if i == 0:
                    for j in range(1, N_DEV):
                        kv_send(j).wait_send()

        acc = [ybuf[0, b].astype(f32) + ybuf[1, b].astype(f32)
               + ybuf[2, b].astype(f32) + ybuf[3, b].astype(f32)
               for b in range(B)]
        for b in range(B):
            out_ref[b] = acc[b]

    return pl.pallas_call(
        body,
        out_shape=jax.ShapeDtypeStruct((B, SQ, D_MODEL), jnp.float32),
        in_specs=[
            pl.BlockSpec(memory_space=pltpu.VMEM),
            pl.BlockSpec(memory_space=pltpu.VMEM),
            pl.BlockSpec(memory_space=pl.ANY),
            pl.BlockSpec(memory_space=pl.ANY),
            pl.BlockSpec(memory_space=pltpu.VMEM),
        ],
        out_specs=pl.BlockSpec(memory_space=pltpu.VMEM),
        scratch_shapes=[
            pltpu.VMEM((B, SKV, N_DEV * HQ, DH), jnp.float32),
            pltpu.VMEM((B, SKV, N_DEV * HQ, DH), jnp.float32),
            pltpu.VMEM((2, B, SKV, HQ, DH), jnp.bfloat16),
            pltpu.VMEM((N_DEV - 1, 2, B, SKV, HQ, DH), jnp.bfloat16),
            pltpu.VMEM((N_DEV, B, SQ, D_MODEL), jnp.bfloat16),
            pltpu.SemaphoreType.DMA((2,)),
            pltpu.SemaphoreType.DMA((N_DEV - 1,)),
            pltpu.SemaphoreType.DMA,
            pltpu.SemaphoreType.DMA((B, N_DEV)),
            pltpu.SemaphoreType.DMA((B, N_DEV)),
        ],
        compiler_params=pltpu.CompilerParams(collective_id=0),
    )(x, Wq, K_ext, V_ext, Wo)
